# baseline (speedup 1.0000x reference)
import functools

import jax
import jax.numpy as jnp
from jax import lax
from jax.experimental import pallas as pl
from jax.experimental.pallas import tpu as pltpu
from jax.experimental.pallas import tpu_sc as plsc

NC = 2
NS = 16
L = 16
NW = NC * NS

B = 16384
D = 32
BPW = B // NW
GN = 8


def _body(uid_hbm, iid_hbm, utT_hbm, itT_hbm, out_hbm,
          uids_v, iids_v, ubufs, ibufs, out_v, sem, sem_i):
    c = lax.axis_index("c")
    s = lax.axis_index("s")
    wid = s * NC + c

    pltpu.sync_copy(uid_hbm.at[wid], uids_v)
    pltpu.sync_copy(iid_hbm.at[wid], iids_v)

    lane = lax.broadcasted_iota(jnp.int32, (L,), 0)
    vq4 = lane & 3
    dgrp = lane >> 2

    def zero(ci, carry):
        out_v[pl.ds(ci * L, L)] = jnp.zeros((L,), jnp.float32)
        return carry

    lax.fori_loop(0, BPW // L, zero, 0)

    HG = 4
    NH = BPW // HG
    RING = GN // HG

    def fire_half(h):
        sb = lax.rem(h, RING) * HG
        gu = plsc.load_gather(uids_v, [h * HG + vq4])
        gi = plsc.load_gather(iids_v, [h * HG + vq4])
        jus = (gu >> 7) * 128
        jis = (gi >> 7) * 128
        for q in range(HG):
            pltpu.async_copy(
                utT_hbm.at[:, pl.ds(pl.multiple_of(jus[q], 128), 128)],
                ubufs.at[sb + q], sem)
            pltpu.async_copy(
                itT_hbm.at[:, pl.ds(pl.multiple_of(jis[q], 128), 128)],
                ibufs.at[sb + q], sem_i)

    def compute_half(h):
        base = h * HG
        sb = lax.rem(h, RING) * HG
        for q in range(HG):
            pltpu.make_async_copy(
                utT_hbm.at[:, pl.ds(0, 128)], ubufs.at[0], sem).wait()
            pltpu.make_async_copy(
                utT_hbm.at[:, pl.ds(0, 128)], ibufs.at[0], sem_i).wait()
        gu = plsc.load_gather(uids_v, [base + vq4])
        gi = plsc.load_gather(iids_v, [base + vq4])
        cu = gu & 127
        ci = gi & 127
        slot = sb + vq4
        acc = jnp.zeros((L,), jnp.float32)
        for k in range(D // 4):
            dk = dgrp * (D // 4) + k
            au = plsc.load_gather(ubufs, [slot, dk, cu])
            ai = plsc.load_gather(ibufs, [slot, dk, ci])
            acc = acc + au * ai
        oidx = base + vq4
        for m in range(4):
            plsc.addupdate_scatter(out_v, [oidx], acc, mask=dgrp == m)

    fire_half(0)

    def half(h, carry):
        fire_half(h + 1)
        compute_half(h)
        return carry

    lax.fori_loop(0, NH - 1, half, 0)
    compute_half(NH - 1)

    pltpu.sync_copy(out_v, out_hbm.at[pl.ds(wid * BPW, BPW)])


def kernel(user_ids, item_ids, user_table, item_table):
    uid = user_ids.astype(jnp.int32).reshape(NW, BPW)
    iid = item_ids.astype(jnp.int32).reshape(NW, BPW)
    utT = user_table.T
    itT = item_table.T
    mesh = plsc.VectorSubcoreMesh(core_axis_name="c", subcore_axis_name="s")
    run = functools.partial(
        pl.kernel,
        mesh=mesh,
        compiler_params=pltpu.CompilerParams(needs_layout_passes=False),
        out_type=jax.ShapeDtypeStruct((B,), jnp.float32),
        scratch_types=[
            pltpu.VMEM((BPW,), jnp.int32),
            pltpu.VMEM((BPW,), jnp.int32),
            pltpu.VMEM((GN, D, 128), jnp.float32),
            pltpu.VMEM((GN, D, 128), jnp.float32),
            pltpu.VMEM((BPW,), jnp.float32),
            pltpu.SemaphoreType.DMA,
            pltpu.SemaphoreType.DMA,
        ],
    )(_body)
    return run(uid, iid, utT, itT)


# speedup vs baseline: 1.1163x; 1.1163x over previous
import functools

import jax
import jax.numpy as jnp
from jax import lax
from jax.experimental import pallas as pl
from jax.experimental.pallas import tpu as pltpu
from jax.experimental.pallas import tpu_sc as plsc

NC = 2
NS = 16
L = 16
NW = NC * NS

B = 16384
D = 32
BPW = B // NW
GN = 8


def _body(uid_hbm, iid_hbm, utT_hbm, itT_hbm, out_hbm,
          uids_v, iids_v, ubufs, ibufs, out_v, sem):
    c = lax.axis_index("c")
    s = lax.axis_index("s")
    wid = s * NC + c

    pltpu.sync_copy(uid_hbm.at[wid], uids_v)
    pltpu.sync_copy(iid_hbm.at[wid], iids_v)

    lane = lax.broadcasted_iota(jnp.int32, (L,), 0)
    vq4 = lane & 3
    dgrp = lane >> 2

    def zero(ci, carry):
        out_v[pl.ds(ci * L, L)] = jnp.zeros((L,), jnp.float32)
        return carry

    lax.fori_loop(0, BPW // L, zero, 0)

    HG = 4
    NH = BPW // HG
    RING = GN // HG

    def fire_half(h):
        sb = lax.rem(h, RING) * HG
        gu = plsc.load_gather(uids_v, [h * HG + vq4])
        gi = plsc.load_gather(iids_v, [h * HG + vq4])
        jus = (gu >> 7) * 128
        jis = (gi >> 7) * 128
        for q in range(HG):
            pltpu.async_copy(
                utT_hbm.at[:, pl.ds(pl.multiple_of(jus[q], 128), 128)],
                ubufs.at[sb + q], sem)
            pltpu.async_copy(
                itT_hbm.at[:, pl.ds(pl.multiple_of(jis[q], 128), 128)],
                ibufs.at[sb + q], sem)

    def compute_half(h):
        base = h * HG
        sb = lax.rem(h, RING) * HG
        for q in range(2 * HG):
            pltpu.make_async_copy(
                utT_hbm.at[:, pl.ds(0, 128)], ubufs.at[0], sem).wait()
        gu = plsc.load_gather(uids_v, [base + vq4])
        gi = plsc.load_gather(iids_v, [base + vq4])
        cu = gu & 127
        ci = gi & 127
        slot = sb + vq4
        acc = jnp.zeros((L,), jnp.float32)
        for k in range(D // 4):
            dk = dgrp * (D // 4) + k
            au = plsc.load_gather(ubufs, [slot, dk, cu])
            ai = plsc.load_gather(ibufs, [slot, dk, ci])
            acc = acc + au * ai
        oidx = base + vq4
        for m in range(4):
            plsc.addupdate_scatter(out_v, [oidx], acc, mask=dgrp == m)

    fire_half(0)

    def half(h, carry):
        fire_half(h + 1)
        compute_half(h)
        return carry

    lax.fori_loop(0, NH - 1, half, 0)
    compute_half(NH - 1)

    pltpu.sync_copy(out_v, out_hbm.at[pl.ds(wid * BPW, BPW)])


def kernel(user_ids, item_ids, user_table, item_table):
    uid = user_ids.astype(jnp.int32).reshape(NW, BPW)
    iid = item_ids.astype(jnp.int32).reshape(NW, BPW)
    utT = user_table.T
    itT = item_table.T
    mesh = plsc.VectorSubcoreMesh(core_axis_name="c", subcore_axis_name="s")
    run = functools.partial(
        pl.kernel,
        mesh=mesh,
        compiler_params=pltpu.CompilerParams(needs_layout_passes=False),
        out_type=jax.ShapeDtypeStruct((B,), jnp.float32),
        scratch_types=[
            pltpu.VMEM((BPW,), jnp.int32),
            pltpu.VMEM((BPW,), jnp.int32),
            pltpu.VMEM((GN, D, 128), jnp.float32),
            pltpu.VMEM((GN, D, 128), jnp.float32),
            pltpu.VMEM((BPW,), jnp.float32),
            pltpu.SemaphoreType.DMA,
        ],
    )(_body)
    return run(uid, iid, utT, itT)
